# baseline (speedup 1.0000x reference)
import functools

import jax
import jax.numpy as jnp
from jax import lax
from jax.experimental import pallas as pl
from jax.experimental.pallas import tpu as pltpu
from jax.experimental.pallas import tpu_sc as plsc

N = 10000
E = 320000
D = 128

NC = 2
NS = 16
NPAD = 10240
HALF = D // NC
ROWS_PT = NPAD // NS
CHUNK = 128
ECHUNKS = 2560
E_PAD = ECHUNKS * CHUNK
CPT = ECHUNKS // NS
HCPT = CPT // 2
CPW = ECHUNKS // (NC * NS)

_mesh = plsc.VectorSubcoreMesh(
    core_axis_name="c", subcore_axis_name="s", num_cores=NC, num_subcores=NS
)


DEGW = 16


@functools.partial(
    pl.kernel,
    out_type=jax.ShapeDtypeStruct((NC, NPAD, DEGW), jnp.float32),
    mesh=_mesh,
    scratch_types=dict(
        dacc=pltpu.VMEM_SHARED((NPAD, DEGW), jnp.float32),
        ones_v=pltpu.VMEM((CHUNK, DEGW), jnp.float32),
        didx=pltpu.VMEM((CPW, CHUNK), jnp.int32),
    ),
    compiler_params=pltpu.CompilerParams(use_tc_tiling_on_sc=False),
)
def _deg_kernel(dst_hbm, zcol_hbm, ones_hbm, out_hbm, dacc, ones_v, didx):
    c = lax.axis_index("c")
    s = lax.axis_index("s")

    pltpu.sync_copy(ones_hbm, ones_v)
    pltpu.sync_copy(zcol_hbm.at[pl.ds(s * ROWS_PT, ROWS_PT)],
                    dacc.at[pl.ds(s * ROWS_PT, ROWS_PT)])
    w = s * NC + c
    pltpu.sync_copy(dst_hbm.at[pl.ds(w * CPW, CPW)], didx)
    plsc.subcore_barrier()

    @pl.loop(0, CPW)
    def _edges(k):
        pltpu.sync_copy(ones_v, dacc.at[didx.at[k]], add=True)

    plsc.subcore_barrier()
    pltpu.sync_copy(
        dacc.at[pl.ds(s * ROWS_PT, ROWS_PT)],
        out_hbm.at[c, pl.ds(s * ROWS_PT, ROWS_PT)],
    )


@functools.partial(
    pl.kernel,
    out_type=jax.ShapeDtypeStruct((NC, NPAD, HALF), jnp.float32),
    mesh=_mesh,
    scratch_types=dict(
        acc=pltpu.VMEM_SHARED((NPAD, HALF), jnp.float32),
        sidx=pltpu.VMEM((CPT, CHUNK), jnp.int32),
        didx=pltpu.VMEM((CPT, CHUNK), jnp.int32),
        rows_a=pltpu.VMEM((CHUNK, HALF), jnp.float32),
        rows_b=pltpu.VMEM((CHUNK, HALF), jnp.float32),
        sem_a=pltpu.SemaphoreType.DMA,
        sem_b=pltpu.SemaphoreType.DMA,
    ),
    compiler_params=pltpu.CompilerParams(use_tc_tiling_on_sc=False),
)
def _edge_kernel(scaled_hbm, src_hbm, dst_hbm, zeros_hbm, out_hbm, acc,
                 sidx, didx, rows_a, rows_b, sem_a, sem_b):
    c = lax.axis_index("c")
    s = lax.axis_index("s")
    r0 = s * ROWS_PT

    pltpu.sync_copy(zeros_hbm.at[pl.ds(r0, ROWS_PT)], acc.at[pl.ds(r0, ROWS_PT)])
    k0 = s * CPT
    pltpu.sync_copy(src_hbm.at[pl.ds(k0, CPT)], sidx)
    pltpu.sync_copy(dst_hbm.at[pl.ds(k0, CPT)], didx)

    plsc.subcore_barrier()

    table = scaled_hbm.at[c]

    def gather(k, rows, sem):
        return pltpu.async_copy(table.at[sidx.at[k]], rows, sem)

    def gather_wait(k, rows, sem):
        pltpu.make_async_copy(table.at[sidx.at[k]], rows, sem).wait()

    def scatter(k, rows):
        pltpu.sync_copy(rows, acc.at[didx.at[k]], add=True)

    gather(0, rows_a, sem_a)

    @pl.loop(0, (CPT - 2) // 2)
    def _pairs(p):
        k = 2 * p
        gather_wait(k, rows_a, sem_a)
        gather(k + 1, rows_b, sem_b)
        scatter(k, rows_a)
        gather_wait(k + 1, rows_b, sem_b)
        gather(k + 2, rows_a, sem_a)
        scatter(k + 1, rows_b)

    gather_wait(CPT - 2, rows_a, sem_a)
    gather(CPT - 1, rows_b, sem_b)
    scatter(CPT - 2, rows_a)
    gather_wait(CPT - 1, rows_b, sem_b)
    scatter(CPT - 1, rows_b)

    plsc.subcore_barrier()
    pltpu.sync_copy(
        acc.at[pl.ds(r0, ROWS_PT)],
        out_hbm.at[c, pl.ds(r0, ROWS_PT)],
    )


BLK = 1280


def _split_store(ref, val):
    ref[0] = val[:, :HALF]
    ref[1] = val[:, HALF:]


def _prep_body(x_ref, w_ref, deg_ref, scaled_ref, dinv_ref):
    mm = jnp.dot(x_ref[...], w_ref[...], preferred_element_type=jnp.float32)
    d = deg_ref[...]
    dinv = lax.rsqrt(1.0 + d[0] + d[1])
    _split_store(scaled_ref, dinv * mm)
    dinv_ref[...] = dinv


_prep_call = pl.pallas_call(
    _prep_body,
    grid=(NPAD // BLK,),
    in_specs=[
        pl.BlockSpec((BLK, D), lambda i: (i, 0)),
        pl.BlockSpec((D, D), lambda i: (0, 0)),
        pl.BlockSpec((NC, BLK, 1), lambda i: (0, i, 0)),
    ],
    out_specs=[
        pl.BlockSpec((NC, BLK, HALF), lambda i: (0, i, 0)),
        pl.BlockSpec((BLK, 1), lambda i: (i, 0)),
    ],
    out_shape=[
        jax.ShapeDtypeStruct((NC, NPAD, HALF), jnp.float32),
        jax.ShapeDtypeStruct((NPAD, 1), jnp.float32),
    ],
)


def _merge_acc(acc_ref):
    a = acc_ref[...]
    return jnp.concatenate([a[0], a[1]], axis=1)


def _mid_body(acc_ref, sc_ref, dinv_ref, b1_ref, w2_ref, out_ref):
    dinv = dinv_ref[...]
    h = jnp.maximum(
        dinv * (_merge_acc(acc_ref) + _merge_acc(sc_ref)) + b1_ref[...], 0.0)
    _split_store(out_ref, dinv * jnp.dot(h, w2_ref[...],
                                         preferred_element_type=jnp.float32))


_mid_call = pl.pallas_call(
    _mid_body,
    grid=(NPAD // BLK,),
    in_specs=[
        pl.BlockSpec((NC, BLK, HALF), lambda i: (0, i, 0)),
        pl.BlockSpec((NC, BLK, HALF), lambda i: (0, i, 0)),
        pl.BlockSpec((BLK, 1), lambda i: (i, 0)),
        pl.BlockSpec((1, D), lambda i: (0, 0)),
        pl.BlockSpec((D, D), lambda i: (0, 0)),
    ],
    out_specs=pl.BlockSpec((NC, BLK, HALF), lambda i: (0, i, 0)),
    out_shape=jax.ShapeDtypeStruct((NC, NPAD, HALF), jnp.float32),
)


def _final_body(acc_ref, sc_ref, dinv_ref, b2_ref, pku_ref, out_ref):
    z = (dinv_ref[...] * (_merge_acc(acc_ref) + _merge_acc(sc_ref))
         + b2_ref[...])
    e = jnp.maximum(z + pku_ref[...], 0.0)
    m = jnp.max(e, axis=1, keepdims=True)
    lse = jnp.log(jnp.sum(jnp.exp(e - m), axis=1, keepdims=True))
    out_ref[...] = e - m - lse


_final_call = pl.pallas_call(
    _final_body,
    grid=(NPAD // BLK,),
    in_specs=[
        pl.BlockSpec((NC, BLK, HALF), lambda i: (0, i, 0)),
        pl.BlockSpec((NC, BLK, HALF), lambda i: (0, i, 0)),
        pl.BlockSpec((BLK, 1), lambda i: (i, 0)),
        pl.BlockSpec((1, D), lambda i: (0, 0)),
        pl.BlockSpec((1, D), lambda i: (0, 0)),
    ],
    out_specs=pl.BlockSpec((BLK, D), lambda i: (i, 0)),
    out_shape=jax.ShapeDtypeStruct((NPAD, D), jnp.float32),
)


def kernel(x, edge_index, W1, b1, W2, b2, P, K, U):
    pad = NPAD - 240 + (jnp.arange(E_PAD - E, dtype=jnp.int32) % 240)
    src = jnp.concatenate([edge_index[0].astype(jnp.int32), pad])
    dst = jnp.concatenate([edge_index[1].astype(jnp.int32), pad])
    src = src.reshape(ECHUNKS, CHUNK)
    dst = dst.reshape(ECHUNKS, CHUNK)
    x_p = jnp.pad(x, ((0, NPAD - N), (0, 0)))

    zcol = jnp.zeros((NPAD, DEGW), jnp.float32)
    ones_c = jnp.ones((CHUNK, DEGW), jnp.float32)
    zeros_half = jnp.zeros((NPAD, HALF), jnp.float32)
    deg = _deg_kernel(dst, zcol, ones_c)[:, :, :1]

    scaled1, dinv = _prep_call(x_p, W1, deg)
    acc1 = _edge_kernel(scaled1, src, dst, zeros_half)
    scaled2 = _mid_call(acc1, scaled1, dinv, b1.reshape(1, D), W2)
    acc2 = _edge_kernel(scaled2, src, dst, zeros_half)
    pku = (P * K * U).reshape(1, D)
    out = _final_call(acc2, scaled2, dinv, b2.reshape(1, D), pku)
    return out[:N]


# speedup vs baseline: 1.0512x; 1.0512x over previous
import functools

import jax
import jax.numpy as jnp
from jax import lax
from jax.experimental import pallas as pl
from jax.experimental.pallas import tpu as pltpu
from jax.experimental.pallas import tpu_sc as plsc

N = 10000
E = 320000
D = 128

NC = 2
NS = 16
NPAD = 10240
HALF = D // NC
ROWS_PT = NPAD // NS
CHUNK = 128
ECHUNKS = 2560
E_PAD = ECHUNKS * CHUNK
CPT = ECHUNKS // NS
HCPT = CPT // 2
CPW = ECHUNKS // (NC * NS)

_mesh = plsc.VectorSubcoreMesh(
    core_axis_name="c", subcore_axis_name="s", num_cores=NC, num_subcores=NS
)


DEGW = 16


@functools.partial(
    pl.kernel,
    out_type=jax.ShapeDtypeStruct((NC, NPAD, DEGW), jnp.float32),
    mesh=_mesh,
    scratch_types=dict(
        dacc=pltpu.VMEM_SHARED((NPAD, DEGW), jnp.float32),
        ones_v=pltpu.VMEM((CHUNK, DEGW), jnp.float32),
        didx=pltpu.VMEM((CPW, CHUNK), jnp.int32),
    ),
    compiler_params=pltpu.CompilerParams(use_tc_tiling_on_sc=False),
)
def _deg_kernel(dst_hbm, zcol_hbm, ones_hbm, out_hbm, dacc, ones_v, didx):
    c = lax.axis_index("c")
    s = lax.axis_index("s")

    pltpu.sync_copy(ones_hbm, ones_v)
    pltpu.sync_copy(zcol_hbm.at[pl.ds(s * ROWS_PT, ROWS_PT)],
                    dacc.at[pl.ds(s * ROWS_PT, ROWS_PT)])
    w = s * NC + c
    pltpu.sync_copy(dst_hbm.at[pl.ds(w * CPW, CPW)], didx)
    plsc.subcore_barrier()

    @pl.loop(0, CPW)
    def _edges(k):
        pltpu.sync_copy(ones_v, dacc.at[didx.at[k]], add=True)

    plsc.subcore_barrier()
    pltpu.sync_copy(
        dacc.at[pl.ds(s * ROWS_PT, ROWS_PT)],
        out_hbm.at[c, pl.ds(s * ROWS_PT, ROWS_PT)],
    )


@functools.partial(
    pl.kernel,
    out_type=jax.ShapeDtypeStruct((NC, NPAD, HALF), jnp.float32),
    mesh=_mesh,
    scratch_types=dict(
        table=pltpu.VMEM_SHARED((NPAD, HALF), jnp.float32),
        acc=pltpu.VMEM_SHARED((NPAD, HALF), jnp.float32),
        sidx=pltpu.VMEM((HCPT, CHUNK), jnp.int32),
        didx=pltpu.VMEM((HCPT, CHUNK), jnp.int32),
        rows_a=pltpu.VMEM((CHUNK, HALF), jnp.float32),
        rows_b=pltpu.VMEM((CHUNK, HALF), jnp.float32),
        sem_a=pltpu.SemaphoreType.DMA,
        sem_b=pltpu.SemaphoreType.DMA,
        sem_sa=pltpu.SemaphoreType.DMA,
        sem_sb=pltpu.SemaphoreType.DMA,
    ),
    compiler_params=pltpu.CompilerParams(use_tc_tiling_on_sc=False),
)
def _edge_kernel(scaled_hbm, src_hbm, dst_hbm, zeros_hbm, out_hbm, table, acc,
                 sidx, didx, rows_a, rows_b, sem_a, sem_b, sem_sa, sem_sb):
    c = lax.axis_index("c")
    s = lax.axis_index("s")
    r0 = s * ROWS_PT

    pltpu.sync_copy(scaled_hbm.at[c, pl.ds(r0, ROWS_PT)],
                    table.at[pl.ds(r0, ROWS_PT)])
    pltpu.sync_copy(zeros_hbm.at[pl.ds(r0, ROWS_PT)], acc.at[pl.ds(r0, ROWS_PT)])
    k0 = s * CPT

    plsc.subcore_barrier()

    def gather(k, rows, sem):
        return pltpu.async_copy(table.at[sidx.at[k]], rows, sem)

    def gather_wait(k, rows, sem):
        pltpu.make_async_copy(table.at[sidx.at[k]], rows, sem).wait()

    def scatter(k, rows, sem):
        return pltpu.async_copy(rows, acc.at[didx.at[k]], sem, add=True)

    def scatter_wait(k, rows, sem):
        pltpu.make_async_copy(rows, acc.at[didx.at[k]], sem).wait()

    for h in range(2):
        pltpu.sync_copy(src_hbm.at[pl.ds(k0 + h * HCPT, HCPT)], sidx)
        pltpu.sync_copy(dst_hbm.at[pl.ds(k0 + h * HCPT, HCPT)], didx)

        gather(0, rows_a, sem_a)
        gather(1, rows_b, sem_b)

        @pl.loop(0, (HCPT - 2) // 2)
        def _pairs(p):
            k = 2 * p
            gather_wait(k, rows_a, sem_a)
            scatter(k, rows_a, sem_sa)
            gather_wait(k + 1, rows_b, sem_b)
            scatter(k + 1, rows_b, sem_sb)
            scatter_wait(k, rows_a, sem_sa)
            gather(k + 2, rows_a, sem_a)
            scatter_wait(k + 1, rows_b, sem_sb)
            gather(k + 3, rows_b, sem_b)

        gather_wait(HCPT - 2, rows_a, sem_a)
        pltpu.sync_copy(rows_a, acc.at[didx.at[HCPT - 2]], add=True)
        gather_wait(HCPT - 1, rows_b, sem_b)
        pltpu.sync_copy(rows_b, acc.at[didx.at[HCPT - 1]], add=True)

    plsc.subcore_barrier()
    pltpu.sync_copy(
        acc.at[pl.ds(r0, ROWS_PT)],
        out_hbm.at[c, pl.ds(r0, ROWS_PT)],
    )


BLK = 1280


def _split_store(ref, val):
    ref[0] = val[:, :HALF]
    ref[1] = val[:, HALF:]


def _prep_body(x_ref, w_ref, deg_ref, scaled_ref, dinv_ref):
    mm = jnp.dot(x_ref[...], w_ref[...], preferred_element_type=jnp.float32)
    d = deg_ref[...]
    dinv = lax.rsqrt(1.0 + d[0] + d[1])
    _split_store(scaled_ref, dinv * mm)
    dinv_ref[...] = dinv


_prep_call = pl.pallas_call(
    _prep_body,
    grid=(NPAD // BLK,),
    in_specs=[
        pl.BlockSpec((BLK, D), lambda i: (i, 0)),
        pl.BlockSpec((D, D), lambda i: (0, 0)),
        pl.BlockSpec((NC, BLK, 1), lambda i: (0, i, 0)),
    ],
    out_specs=[
        pl.BlockSpec((NC, BLK, HALF), lambda i: (0, i, 0)),
        pl.BlockSpec((BLK, 1), lambda i: (i, 0)),
    ],
    out_shape=[
        jax.ShapeDtypeStruct((NC, NPAD, HALF), jnp.float32),
        jax.ShapeDtypeStruct((NPAD, 1), jnp.float32),
    ],
)


def _merge_acc(acc_ref):
    a = acc_ref[...]
    return jnp.concatenate([a[0], a[1]], axis=1)


def _mid_body(acc_ref, sc_ref, dinv_ref, b1_ref, w2_ref, out_ref):
    dinv = dinv_ref[...]
    h = jnp.maximum(
        dinv * (_merge_acc(acc_ref) + _merge_acc(sc_ref)) + b1_ref[...], 0.0)
    _split_store(out_ref, dinv * jnp.dot(h, w2_ref[...],
                                         preferred_element_type=jnp.float32))


_mid_call = pl.pallas_call(
    _mid_body,
    grid=(NPAD // BLK,),
    in_specs=[
        pl.BlockSpec((NC, BLK, HALF), lambda i: (0, i, 0)),
        pl.BlockSpec((NC, BLK, HALF), lambda i: (0, i, 0)),
        pl.BlockSpec((BLK, 1), lambda i: (i, 0)),
        pl.BlockSpec((1, D), lambda i: (0, 0)),
        pl.BlockSpec((D, D), lambda i: (0, 0)),
    ],
    out_specs=pl.BlockSpec((NC, BLK, HALF), lambda i: (0, i, 0)),
    out_shape=jax.ShapeDtypeStruct((NC, NPAD, HALF), jnp.float32),
)


def _final_body(acc_ref, sc_ref, dinv_ref, b2_ref, pku_ref, out_ref):
    z = (dinv_ref[...] * (_merge_acc(acc_ref) + _merge_acc(sc_ref))
         + b2_ref[...])
    e = jnp.maximum(z + pku_ref[...], 0.0)
    m = jnp.max(e, axis=1, keepdims=True)
    lse = jnp.log(jnp.sum(jnp.exp(e - m), axis=1, keepdims=True))
    out_ref[...] = e - m - lse


_final_call = pl.pallas_call(
    _final_body,
    grid=(NPAD // BLK,),
    in_specs=[
        pl.BlockSpec((NC, BLK, HALF), lambda i: (0, i, 0)),
        pl.BlockSpec((NC, BLK, HALF), lambda i: (0, i, 0)),
        pl.BlockSpec((BLK, 1), lambda i: (i, 0)),
        pl.BlockSpec((1, D), lambda i: (0, 0)),
        pl.BlockSpec((1, D), lambda i: (0, 0)),
    ],
    out_specs=pl.BlockSpec((BLK, D), lambda i: (i, 0)),
    out_shape=jax.ShapeDtypeStruct((NPAD, D), jnp.float32),
)


def kernel(x, edge_index, W1, b1, W2, b2, P, K, U):
    pad = NPAD - 240 + (jnp.arange(E_PAD - E, dtype=jnp.int32) % 240)
    src = jnp.concatenate([edge_index[0].astype(jnp.int32), pad])
    dst = jnp.concatenate([edge_index[1].astype(jnp.int32), pad])
    src = src.reshape(ECHUNKS, CHUNK)
    dst = dst.reshape(ECHUNKS, CHUNK)
    x_p = jnp.pad(x, ((0, NPAD - N), (0, 0)))

    zcol = jnp.zeros((NPAD, DEGW), jnp.float32)
    ones_c = jnp.ones((CHUNK, DEGW), jnp.float32)
    zeros_half = jnp.zeros((NPAD, HALF), jnp.float32)
    deg = _deg_kernel(dst, zcol, ones_c)[:, :, :1]

    scaled1, dinv = _prep_call(x_p, W1, deg)
    acc1 = _edge_kernel(scaled1, src, dst, zeros_half)
    scaled2 = _mid_call(acc1, scaled1, dinv, b1.reshape(1, D), W2)
    acc2 = _edge_kernel(scaled2, src, dst, zeros_half)
    pku = (P * K * U).reshape(1, D)
    out = _final_call(acc2, scaled2, dinv, b2.reshape(1, D), pku)
    return out[:N]


# speedup vs baseline: 1.0812x; 1.0286x over previous
import functools

import jax
import jax.numpy as jnp
from jax import lax
from jax.experimental import pallas as pl
from jax.experimental.pallas import tpu as pltpu
from jax.experimental.pallas import tpu_sc as plsc

N = 10000
E = 320000
D = 128

NC = 2
NS = 16
NPAD = 10240
HALF = D // NC
ROWS_PT = NPAD // NS
CHUNK = 128
ECHUNKS = 2560
E_PAD = ECHUNKS * CHUNK
CPT = ECHUNKS // NS
HCPT = CPT // 2
CPW = ECHUNKS // (NC * NS)

_mesh = plsc.VectorSubcoreMesh(
    core_axis_name="c", subcore_axis_name="s", num_cores=NC, num_subcores=NS
)


DEGW = 16


@functools.partial(
    pl.kernel,
    out_type=jax.ShapeDtypeStruct((NC, NPAD, DEGW), jnp.float32),
    mesh=_mesh,
    scratch_types=dict(
        dacc=pltpu.VMEM_SHARED((NPAD, DEGW), jnp.float32),
        ones_v=pltpu.VMEM((CHUNK, DEGW), jnp.float32),
        didx=pltpu.VMEM((CPW, CHUNK), jnp.int32),
    ),
    compiler_params=pltpu.CompilerParams(use_tc_tiling_on_sc=False),
)
def _deg_kernel(dst_hbm, zcol_hbm, ones_hbm, out_hbm, dacc, ones_v, didx):
    c = lax.axis_index("c")
    s = lax.axis_index("s")

    pltpu.sync_copy(ones_hbm, ones_v)
    pltpu.sync_copy(zcol_hbm.at[pl.ds(s * ROWS_PT, ROWS_PT)],
                    dacc.at[pl.ds(s * ROWS_PT, ROWS_PT)])
    w = s * NC + c
    pltpu.sync_copy(dst_hbm.at[pl.ds(w * CPW, CPW)], didx)
    plsc.subcore_barrier()

    @pl.loop(0, CPW)
    def _edges(k):
        pltpu.sync_copy(ones_v, dacc.at[didx.at[k]], add=True)

    plsc.subcore_barrier()
    pltpu.sync_copy(
        dacc.at[pl.ds(s * ROWS_PT, ROWS_PT)],
        out_hbm.at[c, pl.ds(s * ROWS_PT, ROWS_PT)],
    )


@functools.partial(
    pl.kernel,
    out_type=jax.ShapeDtypeStruct((NC, NPAD, HALF), jnp.float32),
    mesh=_mesh,
    scratch_types=dict(
        table=pltpu.VMEM_SHARED((NPAD, HALF), jnp.float32),
        acc=pltpu.VMEM_SHARED((NPAD, HALF), jnp.float32),
        sidx=pltpu.VMEM((HCPT, CHUNK), jnp.int32),
        didx=pltpu.VMEM((HCPT, CHUNK), jnp.int32),
        rows_a=pltpu.VMEM((CHUNK, HALF), jnp.float32),
        rows_b=pltpu.VMEM((CHUNK, HALF), jnp.float32),
        sem_a=pltpu.SemaphoreType.DMA,
        sem_b=pltpu.SemaphoreType.DMA,
    ),
    compiler_params=pltpu.CompilerParams(use_tc_tiling_on_sc=False),
)
def _edge_kernel(scaled_hbm, src_hbm, dst_hbm, zeros_hbm, out_hbm, table, acc,
                 sidx, didx, rows_a, rows_b, sem_a, sem_b):
    c = lax.axis_index("c")
    s = lax.axis_index("s")
    r0 = s * ROWS_PT

    pltpu.sync_copy(scaled_hbm.at[c, pl.ds(r0, ROWS_PT)],
                    table.at[pl.ds(r0, ROWS_PT)])
    pltpu.sync_copy(zeros_hbm.at[pl.ds(r0, ROWS_PT)], acc.at[pl.ds(r0, ROWS_PT)])
    k0 = s * CPT

    plsc.subcore_barrier()

    def gather(k, rows, sem):
        return pltpu.async_copy(table.at[sidx.at[k]], rows, sem)

    def gather_wait(k, rows, sem):
        pltpu.make_async_copy(table.at[sidx.at[k]], rows, sem).wait()

    def scatter(k, rows):
        pltpu.sync_copy(rows, acc.at[didx.at[k]], add=True)

    for h in range(2):
        pltpu.sync_copy(src_hbm.at[pl.ds(k0 + h * HCPT, HCPT)], sidx)
        pltpu.sync_copy(dst_hbm.at[pl.ds(k0 + h * HCPT, HCPT)], didx)

        gather(0, rows_a, sem_a)

        @pl.loop(0, (HCPT - 2) // 2)
        def _pairs(p):
            k = 2 * p
            gather_wait(k, rows_a, sem_a)
            gather(k + 1, rows_b, sem_b)
            scatter(k, rows_a)
            gather_wait(k + 1, rows_b, sem_b)
            gather(k + 2, rows_a, sem_a)
            scatter(k + 1, rows_b)

        gather_wait(HCPT - 2, rows_a, sem_a)
        gather(HCPT - 1, rows_b, sem_b)
        scatter(HCPT - 2, rows_a)
        gather_wait(HCPT - 1, rows_b, sem_b)
        scatter(HCPT - 1, rows_b)

    plsc.subcore_barrier()
    pltpu.sync_copy(
        acc.at[pl.ds(r0, ROWS_PT)],
        out_hbm.at[c, pl.ds(r0, ROWS_PT)],
    )


BLK = 1280


def _split_store(ref, val):
    ref[0] = val[:, :HALF]
    ref[1] = val[:, HALF:]


def _prep_body(x_ref, w_ref, deg_ref, scaled_ref, dinv_ref):
    mm = jnp.dot(x_ref[...], w_ref[...], preferred_element_type=jnp.float32)
    d = deg_ref[...]
    dinv = lax.rsqrt(1.0 + d[0] + d[1])
    _split_store(scaled_ref, dinv * mm)
    dinv_ref[...] = dinv


_prep_call = pl.pallas_call(
    _prep_body,
    grid=(NPAD // BLK,),
    in_specs=[
        pl.BlockSpec((BLK, D), lambda i: (i, 0)),
        pl.BlockSpec((D, D), lambda i: (0, 0)),
        pl.BlockSpec((NC, BLK, 1), lambda i: (0, i, 0)),
    ],
    out_specs=[
        pl.BlockSpec((NC, BLK, HALF), lambda i: (0, i, 0)),
        pl.BlockSpec((BLK, 1), lambda i: (i, 0)),
    ],
    out_shape=[
        jax.ShapeDtypeStruct((NC, NPAD, HALF), jnp.float32),
        jax.ShapeDtypeStruct((NPAD, 1), jnp.float32),
    ],
)


def _merge_acc(acc_ref):
    a = acc_ref[...]
    return jnp.concatenate([a[0], a[1]], axis=1)


def _mid_body(acc_ref, sc_ref, dinv_ref, b1_ref, w2_ref, out_ref):
    dinv = dinv_ref[...]
    h = jnp.maximum(
        dinv * (_merge_acc(acc_ref) + _merge_acc(sc_ref)) + b1_ref[...], 0.0)
    _split_store(out_ref, dinv * jnp.dot(h, w2_ref[...],
                                         preferred_element_type=jnp.float32))


_mid_call = pl.pallas_call(
    _mid_body,
    grid=(NPAD // BLK,),
    in_specs=[
        pl.BlockSpec((NC, BLK, HALF), lambda i: (0, i, 0)),
        pl.BlockSpec((NC, BLK, HALF), lambda i: (0, i, 0)),
        pl.BlockSpec((BLK, 1), lambda i: (i, 0)),
        pl.BlockSpec((1, D), lambda i: (0, 0)),
        pl.BlockSpec((D, D), lambda i: (0, 0)),
    ],
    out_specs=pl.BlockSpec((NC, BLK, HALF), lambda i: (0, i, 0)),
    out_shape=jax.ShapeDtypeStruct((NC, NPAD, HALF), jnp.float32),
)


def _final_body(acc_ref, sc_ref, dinv_ref, b2_ref, pku_ref, out_ref):
    z = (dinv_ref[...] * (_merge_acc(acc_ref) + _merge_acc(sc_ref))
         + b2_ref[...])
    e = jnp.maximum(z + pku_ref[...], 0.0)
    m = jnp.max(e, axis=1, keepdims=True)
    lse = jnp.log(jnp.sum(jnp.exp(e - m), axis=1, keepdims=True))
    out_ref[...] = e - m - lse


_final_call = pl.pallas_call(
    _final_body,
    grid=(NPAD // BLK,),
    in_specs=[
        pl.BlockSpec((NC, BLK, HALF), lambda i: (0, i, 0)),
        pl.BlockSpec((NC, BLK, HALF), lambda i: (0, i, 0)),
        pl.BlockSpec((BLK, 1), lambda i: (i, 0)),
        pl.BlockSpec((1, D), lambda i: (0, 0)),
        pl.BlockSpec((1, D), lambda i: (0, 0)),
    ],
    out_specs=pl.BlockSpec((BLK, D), lambda i: (i, 0)),
    out_shape=jax.ShapeDtypeStruct((NPAD, D), jnp.float32),
)


def kernel(x, edge_index, W1, b1, W2, b2, P, K, U):
    pad = NPAD - 240 + (jnp.arange(E_PAD - E, dtype=jnp.int32) % 240)
    src = jnp.concatenate([edge_index[0].astype(jnp.int32), pad])
    dst = jnp.concatenate([edge_index[1].astype(jnp.int32), pad])
    src = src.reshape(ECHUNKS, CHUNK)
    dst = dst.reshape(ECHUNKS, CHUNK)
    x_p = jnp.pad(x, ((0, NPAD - N), (0, 0)))

    zcol = jnp.zeros((NPAD, DEGW), jnp.float32)
    ones_c = jnp.ones((CHUNK, DEGW), jnp.float32)
    zeros_half = jnp.zeros((NPAD, HALF), jnp.float32)
    deg = _deg_kernel(dst, zcol, ones_c)[:, :, :1]

    scaled1, dinv = _prep_call(x_p, W1, deg)
    acc1 = _edge_kernel(scaled1, src, dst, zeros_half)
    scaled2 = _mid_call(acc1, scaled1, dinv, b1.reshape(1, D), W2)
    acc2 = _edge_kernel(scaled2, src, dst, zeros_half)
    pku = (P * K * U).reshape(1, D)
    out = _final_call(acc2, scaled2, dinv, b2.reshape(1, D), pku)
    return out[:N]
